# baseline (speedup 1.0000x reference)
import functools
import jax
import jax.numpy as jnp
from jax import lax
from jax.experimental import pallas as pl
from jax.experimental.pallas import tpu as pltpu
from jax.experimental.pallas import tpu_sc as plsc

_VOCAB = 1000000
_D = 32
_B = 16384
_A = 50

_NC = 2
_NS = 16
_NW = _NC * _NS

_B_PER_W = _B // _NW
_BG = 64
_N_BG = _B_PER_W // _BG
_AG = 10
_NG = _A // _AG
_ROWS = _AG * _BG

_mesh = plsc.VectorSubcoreMesh(core_axis_name="c", subcore_axis_name="s")


@functools.partial(
    pl.kernel,
    mesh=_mesh,
    out_type=[
        jax.ShapeDtypeStruct((_D, _B), jnp.float32),
        jax.ShapeDtypeStruct((_A, _D, _B), jnp.float32),
    ],
    scratch_types=[
        pltpu.VMEM((_BG, _A), jnp.int32),
        pltpu.VMEM((_BG, 1), jnp.int32),
        pltpu.VMEM((_BG,), jnp.int32),
        pltpu.VMEM((_BG, _D), jnp.float32),
        pltpu.VMEM((_D, _BG + 1), jnp.float32),
        pltpu.VMEM((_ROWS,), jnp.int32),
        pltpu.VMEM((_ROWS,), jnp.int32),
        pltpu.VMEM((_ROWS, _D), jnp.float32),
        pltpu.VMEM((_ROWS, _D), jnp.float32),
        pltpu.VMEM((_AG, _D, _BG + 1), jnp.float32),
        pltpu.SemaphoreType.DMA,
        pltpu.SemaphoreType.DMA,
        pltpu.SemaphoreType.DMA,
    ],
    compiler_params=pltpu.CompilerParams(
        use_tc_tiling_on_sc=False, needs_layout_passes=False,
        disable_bounds_checks=True),
)
def _gather_kernel(table_hbm, obs_idx_hbm, act_idx_hbm, obs_out, act_out,
                   idxblk, oidx2, olist, orows, otr,
                   alist0, alist1, grows0, grows1, tslab,
                   osem, gsem0, gsem1):
    wid = lax.axis_index("s") * _NC + lax.axis_index("c")
    iota = lax.iota(jnp.int32, 16)
    iota16 = iota + 16
    zeros16 = jnp.zeros((16,), jnp.int32)
    alist = (alist0, alist1)
    grows = (grows0, grows1)
    gsem = (gsem0, gsem1)

    def build_group(g, buf):
        a0 = g * _AG

        def ai_body(ai, c):
            cols = jnp.full((16,), a0 + ai, jnp.int32)
            for k in range(_BG // 16):
                v = plsc.load_gather(idxblk, [iota + (k * 16), cols])
                alist[buf][pl.ds(ai * _BG + k * 16, 16)] = v
            return c

        lax.fori_loop(0, _AG, ai_body, 0)

    def transpose_group(g, buf, b0):
        def ai_body(ai, c):
            base = ai * _BG
            tsl = tslab.at[ai]

            def j8_body(j8, c2):
                j0 = base + j8 * 8
                for u in range(8):
                    j = j0 + u
                    jcol = jnp.full((16,), j8 * 8 + u, jnp.int32)
                    v0 = grows[buf][j, pl.ds(0, 16)]
                    v1 = grows[buf][j, pl.ds(16, 16)]
                    plsc.store_scatter(tsl, [iota, jcol], v0)
                    plsc.store_scatter(tsl, [iota16, jcol], v1)
                return c2

            lax.fori_loop(0, _BG // 8, j8_body, 0)
            return c

        lax.fori_loop(0, _AG, ai_body, 0)
        pltpu.sync_copy(
            tslab.at[:, :, pl.ds(0, _BG)],
            act_out.at[pl.ds(g * _AG, _AG), :, pl.ds(b0, _BG)])

    def bg_body(bg, carry):
        b0 = wid * _B_PER_W + bg * _BG

        pltpu.sync_copy(act_idx_hbm.at[pl.ds(b0, _BG)], idxblk)
        pltpu.sync_copy(obs_idx_hbm.at[pl.ds(b0, _BG)], oidx2)

        for k in range(_BG // 16):
            v = plsc.load_gather(oidx2, [iota + (k * 16), zeros16])
            olist[pl.ds(k * 16, 16)] = v
        obs_cp = pltpu.async_copy(table_hbm.at[olist], orows, osem)

        build_group(0, 0)
        cps = [pltpu.async_copy(table_hbm.at[alist[0]], grows[0], gsem[0]),
               None]
        for g in range(1, _NG):
            buf = g % 2
            build_group(g, buf)
            cps[buf] = pltpu.async_copy(
                table_hbm.at[alist[buf]], grows[buf], gsem[buf])
            cps[1 - buf].wait()
            transpose_group(g - 1, 1 - buf, b0)
        cps[(_NG - 1) % 2].wait()
        transpose_group(_NG - 1, (_NG - 1) % 2, b0)

        obs_cp.wait()

        def oj_body(j8, c2):
            for u in range(8):
                j = j8 * 8 + u
                jcol = jnp.full((16,), j, jnp.int32)
                v0 = orows[j, pl.ds(0, 16)]
                v1 = orows[j, pl.ds(16, 16)]
                plsc.store_scatter(otr, [iota, jcol], v0)
                plsc.store_scatter(otr, [iota16, jcol], v1)
            return c2

        lax.fori_loop(0, _BG // 8, oj_body, 0)
        pltpu.sync_copy(otr.at[:, pl.ds(0, _BG)], obs_out.at[:, pl.ds(b0, _BG)])
        return carry

    lax.fori_loop(0, _N_BG, bg_body, 0)


_DEPAD_R = 8000


def _depad_body(in_ref, out_ref):
    x = in_ref[...].reshape(_DEPAD_R // 4, 4, _D)
    parts = [x[:, k, :] for k in range(4)]
    y = jnp.concatenate(parts, axis=1)
    out_ref[...] = y.reshape(_DEPAD_R * _D)


_depad = pl.pallas_call(
    _depad_body,
    grid=(_VOCAB // _DEPAD_R,),
    in_specs=[pl.BlockSpec((_DEPAD_R, _D), lambda i: (i, 0))],
    out_specs=pl.BlockSpec((_DEPAD_R * _D,), lambda i: (i,)),
    out_shape=jax.ShapeDtypeStruct((_VOCAB * _D,), jnp.float32),
)


def kernel(table, action_mask, sub_index, derived_sub_indices):
    obs_idx = sub_index.astype(jnp.int32)
    act_idx = derived_sub_indices.astype(jnp.int32)
    table_lin = _depad(table).reshape(_VOCAB, _D)
    obs_db, act_adb = _gather_kernel(table_lin, obs_idx, act_idx)
    return (jnp.transpose(obs_db, (1, 0)),
            jnp.transpose(act_adb, (2, 0, 1)),
            action_mask)


# speedup vs baseline: 1.2611x; 1.2611x over previous
import functools
import jax
import jax.numpy as jnp
from jax import lax
from jax.experimental import pallas as pl
from jax.experimental.pallas import tpu as pltpu
from jax.experimental.pallas import tpu_sc as plsc

_VOCAB = 1000000
_D = 32
_B = 16384
_A = 50

_NC = 2
_NS = 16
_NW = _NC * _NS

_B_PER_W = _B // _NW
_BG = 64
_N_BG = _B_PER_W // _BG
_AG = 10
_NG = _A // _AG
_ROWS = _AG * _BG

_mesh = plsc.VectorSubcoreMesh(core_axis_name="c", subcore_axis_name="s")


@functools.partial(
    pl.kernel,
    mesh=_mesh,
    out_type=[
        jax.ShapeDtypeStruct((_D, _B), jnp.float32),
        jax.ShapeDtypeStruct((_A, _D, _B), jnp.float32),
    ],
    scratch_types=[
        pltpu.VMEM((_BG, _A), jnp.int32),
        pltpu.VMEM((_BG, 1), jnp.int32),
        pltpu.VMEM((_BG,), jnp.int32),
        pltpu.VMEM((_BG, _D), jnp.float32),
        pltpu.VMEM((_D, _BG + 1), jnp.float32),
        pltpu.VMEM((_ROWS,), jnp.int32),
        pltpu.VMEM((_ROWS,), jnp.int32),
        pltpu.VMEM((_ROWS, _D), jnp.float32),
        pltpu.VMEM((_ROWS, _D), jnp.float32),
        pltpu.VMEM((_AG, _D, _BG + 1), jnp.float32),
        pltpu.SemaphoreType.DMA,
        pltpu.SemaphoreType.DMA,
        pltpu.SemaphoreType.DMA,
    ],
    compiler_params=pltpu.CompilerParams(
        use_tc_tiling_on_sc=False, needs_layout_passes=False,
        disable_bounds_checks=True),
)
def _gather_kernel(table_hbm, obs_idx_hbm, act_idx_hbm, obs_out, act_out,
                   idxblk, oidx2, olist, orows, otr,
                   alist0, alist1, grows0, grows1, tslab,
                   osem, gsem0, gsem1):
    wid = lax.axis_index("s") * _NC + lax.axis_index("c")
    iota = lax.iota(jnp.int32, 16)
    iota16 = iota + 16
    zeros16 = jnp.zeros((16,), jnp.int32)
    alist = (alist0, alist1)
    grows = (grows0, grows1)
    gsem = (gsem0, gsem1)

    def build_group(g, buf):
        a0 = g * _AG

        def ai_body(ai, c):
            cols = jnp.full((16,), a0 + ai, jnp.int32)
            for k in range(_BG // 16):
                v = plsc.load_gather(idxblk, [iota + (k * 16), cols])
                alist[buf][pl.ds(ai * _BG + k * 16, 16)] = v
            return c

        lax.fori_loop(0, _AG, ai_body, 0)

    def transpose_group(g, buf, b0):
        def ai_body(ai, c):
            base = ai * _BG
            tsl = tslab.at[ai]

            def j8_body(j8, c2):
                j0 = base + j8 * 8
                for u in range(8):
                    j = j0 + u
                    jcol = jnp.full((16,), j8 * 8 + u, jnp.int32)
                    v0 = grows[buf][j, pl.ds(0, 16)]
                    v1 = grows[buf][j, pl.ds(16, 16)]
                    plsc.store_scatter(tsl, [iota, jcol], v0)
                    plsc.store_scatter(tsl, [iota16, jcol], v1)
                return c2

            lax.fori_loop(0, _BG // 8, j8_body, 0)
            return c

        lax.fori_loop(0, _AG, ai_body, 0)
        pltpu.sync_copy(
            tslab.at[:, :, pl.ds(0, _BG)],
            act_out.at[pl.ds(g * _AG, _AG), :, pl.ds(b0, _BG)])

    def bg_body(bg, carry):
        b0 = wid * _B_PER_W + bg * _BG

        pltpu.sync_copy(act_idx_hbm.at[pl.ds(b0, _BG)], idxblk)
        pltpu.sync_copy(obs_idx_hbm.at[pl.ds(b0, _BG)], oidx2)

        for k in range(_BG // 16):
            v = plsc.load_gather(oidx2, [iota + (k * 16), zeros16])
            olist[pl.ds(k * 16, 16)] = v
        obs_cp = pltpu.async_copy(table_hbm.at[olist], orows, osem)

        build_group(0, 0)
        cps = [pltpu.async_copy(table_hbm.at[alist[0]], grows[0], gsem[0]),
               None]
        for g in range(1, _NG):
            buf = g % 2
            build_group(g, buf)
            cps[buf] = pltpu.async_copy(
                table_hbm.at[alist[buf]], grows[buf], gsem[buf])
            cps[1 - buf].wait()
            transpose_group(g - 1, 1 - buf, b0)
        cps[(_NG - 1) % 2].wait()
        transpose_group(_NG - 1, (_NG - 1) % 2, b0)

        obs_cp.wait()

        def oj_body(j8, c2):
            for u in range(8):
                j = j8 * 8 + u
                jcol = jnp.full((16,), j, jnp.int32)
                v0 = orows[j, pl.ds(0, 16)]
                v1 = orows[j, pl.ds(16, 16)]
                plsc.store_scatter(otr, [iota, jcol], v0)
                plsc.store_scatter(otr, [iota16, jcol], v1)
            return c2

        lax.fori_loop(0, _BG // 8, oj_body, 0)
        pltpu.sync_copy(otr.at[:, pl.ds(0, _BG)], obs_out.at[:, pl.ds(b0, _BG)])
        return carry

    lax.fori_loop(0, _N_BG, bg_body, 0)


_TT_C = 8192


def _tt_body(in_ref, out_ref):
    xt = jnp.transpose(in_ref[...], (1, 0))
    x3 = xt.reshape(_TT_C // 4, 4, _D)
    y = jnp.concatenate([x3[:, k, :] for k in range(4)], axis=1)
    out_ref[...] = y.reshape(_TT_C * _D)


_table_to_rows = pl.pallas_call(
    _tt_body,
    grid=(pl.cdiv(_VOCAB, _TT_C),),
    in_specs=[pl.BlockSpec((_D, _TT_C), lambda i: (0, i))],
    out_specs=pl.BlockSpec((_TT_C * _D,), lambda i: (i,)),
    out_shape=jax.ShapeDtypeStruct((_VOCAB * _D,), jnp.float32),
)


def kernel(table, action_mask, sub_index, derived_sub_indices):
    obs_idx = sub_index.astype(jnp.int32)
    act_idx = derived_sub_indices.astype(jnp.int32)
    table_lin = _table_to_rows(jnp.transpose(table)).reshape(_VOCAB, _D)
    obs_db, act_adb = _gather_kernel(table_lin, obs_idx, act_idx)
    return (jnp.transpose(obs_db, (1, 0)),
            jnp.transpose(act_adb, (2, 0, 1)),
            action_mask)


# speedup vs baseline: 1.2720x; 1.0086x over previous
import functools
import jax
import jax.numpy as jnp
from jax import lax
from jax.experimental import pallas as pl
from jax.experimental.pallas import tpu as pltpu
from jax.experimental.pallas import tpu_sc as plsc

_VOCAB = 1000000
_D = 32
_B = 16384
_A = 50

_NC = 2
_NS = 16
_NW = _NC * _NS

_B_PER_W = _B // _NW
_BG = 64
_N_BG = _B_PER_W // _BG
_AG = 10
_NG = _A // _AG
_ROWS = _AG * _BG

_mesh = plsc.VectorSubcoreMesh(core_axis_name="c", subcore_axis_name="s")


@functools.partial(
    pl.kernel,
    mesh=_mesh,
    out_type=[
        jax.ShapeDtypeStruct((_D, _B), jnp.float32),
        jax.ShapeDtypeStruct((_A, _D, _B), jnp.float32),
    ],
    scratch_types=[
        pltpu.VMEM((_BG, _A), jnp.int32),
        pltpu.VMEM((_BG, 1), jnp.int32),
        pltpu.VMEM((_BG,), jnp.int32),
        pltpu.VMEM((_BG, _D), jnp.float32),
        pltpu.VMEM((_D, _BG + 1), jnp.float32),
        pltpu.VMEM((_ROWS,), jnp.int32),
        pltpu.VMEM((_ROWS,), jnp.int32),
        pltpu.VMEM((_ROWS, _D), jnp.float32),
        pltpu.VMEM((_ROWS, _D), jnp.float32),
        pltpu.VMEM((_AG, _D, _BG + 1), jnp.float32),
        pltpu.SemaphoreType.DMA,
        pltpu.SemaphoreType.DMA,
        pltpu.SemaphoreType.DMA,
    ],
    compiler_params=pltpu.CompilerParams(
        use_tc_tiling_on_sc=False, needs_layout_passes=False,
        disable_bounds_checks=True),
)
def _gather_kernel(table_hbm, obs_idx_hbm, act_idx_hbm, obs_out, act_out,
                   idxblk, oidx2, olist, orows, otr,
                   alist0, alist1, grows0, grows1, tslab,
                   osem, gsem0, gsem1):
    wid = lax.axis_index("s") * _NC + lax.axis_index("c")
    iota = lax.iota(jnp.int32, 16)
    iota16 = iota + 16
    zeros16 = jnp.zeros((16,), jnp.int32)
    alist = (alist0, alist1)
    grows = (grows0, grows1)
    gsem = (gsem0, gsem1)

    def build_group(g, buf):
        a0 = g * _AG

        def ai_body(ai, c):
            cols = jnp.full((16,), a0 + ai, jnp.int32)
            for k in range(_BG // 16):
                v = plsc.load_gather(idxblk, [iota + (k * 16), cols])
                alist[buf][pl.ds(ai * _BG + k * 16, 16)] = v
            return c

        lax.fori_loop(0, _AG, ai_body, 0)

    def transpose_group(g, buf, b0):
        def ai_body(ai, c):
            base = ai * _BG
            tsl = tslab.at[ai]

            def j8_body(j8, c2):
                j0 = base + j8 * 8
                for u in range(8):
                    j = j0 + u
                    jcol = jnp.full((16,), j8 * 8 + u, jnp.int32)
                    v0 = grows[buf][j, pl.ds(0, 16)]
                    v1 = grows[buf][j, pl.ds(16, 16)]
                    plsc.store_scatter(tsl, [iota, jcol], v0)
                    plsc.store_scatter(tsl, [iota16, jcol], v1)
                return c2

            lax.fori_loop(0, _BG // 8, j8_body, 0)
            return c

        lax.fori_loop(0, _AG, ai_body, 0)
        pltpu.sync_copy(
            tslab.at[:, :, pl.ds(0, _BG)],
            act_out.at[pl.ds(g * _AG, _AG), :, pl.ds(b0, _BG)])

    def bg_body(bg, carry):
        b0 = wid * _B_PER_W + bg * _BG

        pltpu.sync_copy(act_idx_hbm.at[pl.ds(b0, _BG)], idxblk)
        pltpu.sync_copy(obs_idx_hbm.at[pl.ds(b0, _BG)], oidx2)

        for k in range(_BG // 16):
            v = plsc.load_gather(oidx2, [iota + (k * 16), zeros16])
            olist[pl.ds(k * 16, 16)] = v
        obs_cp = pltpu.async_copy(table_hbm.at[olist], orows, osem)

        build_group(0, 0)
        cps = [pltpu.async_copy(table_hbm.at[alist[0]], grows[0], gsem[0]),
               None]
        for g in range(1, _NG):
            buf = g % 2
            build_group(g, buf)
            cps[buf] = pltpu.async_copy(
                table_hbm.at[alist[buf]], grows[buf], gsem[buf])
            cps[1 - buf].wait()
            transpose_group(g - 1, 1 - buf, b0)
        cps[(_NG - 1) % 2].wait()
        transpose_group(_NG - 1, (_NG - 1) % 2, b0)

        obs_cp.wait()

        def oj_body(j8, c2):
            for u in range(8):
                j = j8 * 8 + u
                jcol = jnp.full((16,), j, jnp.int32)
                v0 = orows[j, pl.ds(0, 16)]
                v1 = orows[j, pl.ds(16, 16)]
                plsc.store_scatter(otr, [iota, jcol], v0)
                plsc.store_scatter(otr, [iota16, jcol], v1)
            return c2

        lax.fori_loop(0, _BG // 8, oj_body, 0)
        pltpu.sync_copy(otr.at[:, pl.ds(0, _BG)], obs_out.at[:, pl.ds(b0, _BG)])
        return carry

    lax.fori_loop(0, _N_BG, bg_body, 0)


_TT_C = 16384


def _tt_body(in_ref, out_ref):
    xt = jnp.transpose(in_ref[...], (1, 0))
    x3 = xt.reshape(_TT_C // 4, 4, _D)
    y = jnp.concatenate([x3[:, k, :] for k in range(4)], axis=1)
    out_ref[...] = y.reshape(_TT_C * _D)


_table_to_rows = pl.pallas_call(
    _tt_body,
    grid=(pl.cdiv(_VOCAB, _TT_C),),
    in_specs=[pl.BlockSpec((_D, _TT_C), lambda i: (0, i))],
    out_specs=pl.BlockSpec((_TT_C * _D,), lambda i: (i,)),
    out_shape=jax.ShapeDtypeStruct((_VOCAB * _D,), jnp.float32),
)


def kernel(table, action_mask, sub_index, derived_sub_indices):
    obs_idx = sub_index.astype(jnp.int32)
    act_idx = derived_sub_indices.astype(jnp.int32)
    table_lin = _table_to_rows(jnp.transpose(table)).reshape(_VOCAB, _D)
    obs_db, act_adb = _gather_kernel(table_lin, obs_idx, act_idx)
    return (jnp.transpose(obs_db, (1, 0)),
            jnp.transpose(act_adb, (2, 0, 1)),
            action_mask)
